# baseline (speedup 1.0000x reference)
import functools

import jax
import jax.numpy as jnp
from jax import lax
from jax.experimental import pallas as pl
from jax.experimental.pallas import tpu as pltpu
from jax.experimental.pallas import tpu_sc as plsc

L = 16
NC = 2
NS = 16
NW = NC * NS
S = 4096
T = 200
D = 64
AB = S // NW
SCALE = 8.0

_mesh = plsc.VectorSubcoreMesh(core_axis_name="c", subcore_axis_name="s")


@functools.partial(
    pl.kernel,
    mesh=_mesh,
    out_type=jax.ShapeDtypeStruct((T, D, S), jnp.float32),
    scratch_types=[
        pltpu.VMEM((T, AB), jnp.int32),
        pltpu.VMEM((AB, D), jnp.float32),
        pltpu.VMEM((AB, D), jnp.float32),
        pltpu.VMEM((D, AB), jnp.float32),
        pltpu.VMEM((D, AB), jnp.float32),
        pltpu.SemaphoreType.DMA,
        pltpu.SemaphoreType.DMA,
        pltpu.SemaphoreType.DMA,
        pltpu.SemaphoreType.DMA,
    ],
    compiler_params=pltpu.CompilerParams(
        use_tc_tiling_on_sc=False, needs_layout_passes=False),
)
def _embed_lookup(table_hbm, idxT_hbm, out_hbm, idx_v, rows0, rows1,
                  tout0, tout1, g0, g1, o0, o1):
    wid = lax.axis_index("s") * NC + lax.axis_index("c")
    a0 = wid * AB
    rows = (rows0, rows1)
    tout = (tout0, tout1)
    gsem = (g0, g1)
    osem = (o0, o1)

    pltpu.sync_copy(idxT_hbm.at[:, pl.ds(a0, AB)], idx_v)

    iota16 = lax.broadcasted_iota(jnp.int32, (L,), 0)

    def start_gather(b, p):
        pltpu.async_copy(table_hbm.at[idx_v.at[b]], rows[p], gsem[p])

    def wait_gather(b, p):
        pltpu.make_async_copy(
            table_hbm.at[idx_v.at[b]], rows[p], gsem[p]).wait()

    def start_out(b, p):
        pltpu.async_copy(tout[p], out_hbm.at[b, :, pl.ds(a0, AB)], osem[p])

    def wait_out(b, p):
        pltpu.make_async_copy(
            tout[p], out_hbm.at[b, :, pl.ds(a0, AB)], osem[p]).wait()

    def transpose_scale(p):
        def body(i, carry):
            for u in range(4):
                a = i * 4 + u
                cols = jnp.full((L,), a, dtype=jnp.int32)
                for j in range(D // L):
                    vals = rows[p][a, pl.ds(j * L, L)] * SCALE
                    plsc.store_scatter(
                        tout[p], [j * L + iota16, cols], vals)
            return carry
        lax.fori_loop(0, AB // 4, body, 0)

    start_gather(0, 0)
    start_gather(1, 1)

    wait_gather(0, 0)
    transpose_scale(0)
    start_out(0, 0)
    wait_gather(1, 1)
    transpose_scale(1)
    start_out(1, 1)

    def superstep(k, carry):
        b0 = 2 * k
        b1 = 2 * k + 1
        wait_out(b0, 0)
        start_gather(b0, 0)
        wait_out(b1, 1)
        start_gather(b1, 1)
        wait_gather(b0, 0)
        transpose_scale(0)
        start_out(b0, 0)
        wait_gather(b1, 1)
        transpose_scale(1)
        start_out(b1, 1)
        return carry

    lax.fori_loop(1, T // 2, superstep, 0)
    wait_out(T - 2, 0)
    wait_out(T - 1, 1)


def kernel(token_ids, embed_table):
    idxT = token_ids.T
    outT = _embed_lookup(embed_table, idxT)
    return outT.transpose(2, 0, 1)


# speedup vs baseline: 1.6490x; 1.6490x over previous
import functools

import jax
import jax.numpy as jnp
from jax import lax
from jax.experimental import pallas as pl
from jax.experimental.pallas import tpu as pltpu
from jax.experimental.pallas import tpu_sc as plsc

L = 16
NC = 2
NS = 16
NW = NC * NS
S = 4096
T = 200
D = 64
AB = S // NW
TP = AB + 1
SCALE = 8.0

_mesh = plsc.VectorSubcoreMesh(core_axis_name="c", subcore_axis_name="s")


@functools.partial(
    pl.kernel,
    mesh=_mesh,
    out_type=jax.ShapeDtypeStruct((T, D // 8, S // AB, 8, AB), jnp.float32),
    scratch_types=[
        pltpu.VMEM((T, AB), jnp.int32),
        pltpu.VMEM((AB, D), jnp.float32),
        pltpu.VMEM((AB, D), jnp.float32),
        pltpu.VMEM((D, TP), jnp.float32),
        pltpu.VMEM((D, TP), jnp.float32),
        pltpu.SemaphoreType.DMA,
        pltpu.SemaphoreType.DMA,
        pltpu.SemaphoreType.DMA,
        pltpu.SemaphoreType.DMA,
    ],
    compiler_params=pltpu.CompilerParams(
        use_tc_tiling_on_sc=False, needs_layout_passes=False),
)
def _embed_lookup(table_hbm, idxT_hbm, out_hbm, idx_v, rows0, rows1,
                  tout0, tout1, g0, g1, o0, o1):
    wid = lax.axis_index("s") * NC + lax.axis_index("c")
    a0 = wid * AB
    rows = (rows0, rows1)
    tout = (tout0, tout1)
    gsem = (g0, g1)
    osem = (o0, o1)

    pltpu.sync_copy(idxT_hbm.at[:, pl.ds(a0, AB)], idx_v)

    iota16 = lax.broadcasted_iota(jnp.int32, (L,), 0)

    def start_gather(b, p):
        pltpu.async_copy(table_hbm.at[idx_v.at[b]], rows[p], gsem[p])

    def wait_gather(b, p):
        pltpu.make_async_copy(
            table_hbm.at[idx_v.at[b]], rows[p], gsem[p]).wait()

    def start_out(b, p):
        for dh in range(D // 8):
            pltpu.async_copy(
                tout[p].at[pl.ds(dh * 8, 8), pl.ds(0, AB)],
                out_hbm.at[b, dh, wid], osem[p])

    def wait_out(b, p):
        for dh in range(D // 8):
            pltpu.make_async_copy(
                tout[p].at[pl.ds(dh * 8, 8), pl.ds(0, AB)],
                out_hbm.at[b, dh, wid], osem[p]).wait()

    def transpose_scale(p):
        def body(i, carry):
            for u in range(4):
                a = i * 4 + u
                cols = jnp.full((L,), a, dtype=jnp.int32)
                for j in range(D // L):
                    vals = rows[p][a, pl.ds(j * L, L)] * SCALE
                    plsc.store_scatter(
                        tout[p], [j * L + iota16, cols], vals)
            return carry
        lax.fori_loop(0, AB // 4, body, 0)

    start_gather(0, 0)
    start_gather(1, 1)

    wait_gather(0, 0)
    transpose_scale(0)
    start_out(0, 0)
    wait_gather(1, 1)
    transpose_scale(1)
    start_out(1, 1)

    def superstep(k, carry):
        b0 = 2 * k
        b1 = 2 * k + 1
        wait_out(b0, 0)
        start_gather(b0, 0)
        wait_out(b1, 1)
        start_gather(b1, 1)
        wait_gather(b0, 0)
        transpose_scale(0)
        start_out(b0, 0)
        wait_gather(b1, 1)
        transpose_scale(1)
        start_out(b1, 1)
        return carry

    lax.fori_loop(1, T // 2, superstep, 0)
    wait_out(T - 2, 0)
    wait_out(T - 1, 1)


def kernel(token_ids, embed_table):
    idxT = token_ids.T
    out5 = _embed_lookup(embed_table, idxT)
    out = out5.transpose(2, 4, 0, 1, 3).reshape(S, T, D)
    return out


# speedup vs baseline: 2.0788x; 1.2606x over previous
import functools

import jax
import jax.numpy as jnp
from jax import lax
from jax.experimental import pallas as pl
from jax.experimental.pallas import tpu as pltpu
from jax.experimental.pallas import tpu_sc as plsc

L = 16
NC = 2
NS = 16
NW = NC * NS
S = 4096
T = 200
D = 64
AB = S // NW
TP = AB + 1
SCALE = 8.0

_mesh = plsc.VectorSubcoreMesh(core_axis_name="c", subcore_axis_name="s")


@functools.partial(
    pl.kernel,
    mesh=_mesh,
    out_type=jax.ShapeDtypeStruct((T, D // 8, S // AB, 8, AB), jnp.float32),
    scratch_types=[
        pltpu.VMEM((T, AB), jnp.int32),
        pltpu.VMEM((AB, D), jnp.float32),
        pltpu.VMEM((AB, D), jnp.float32),
        pltpu.VMEM((D // 8, 8, TP), jnp.float32),
        pltpu.VMEM((D // 8, 8, TP), jnp.float32),
        pltpu.SemaphoreType.DMA,
        pltpu.SemaphoreType.DMA,
        pltpu.SemaphoreType.DMA,
        pltpu.SemaphoreType.DMA,
    ],
    compiler_params=pltpu.CompilerParams(
        use_tc_tiling_on_sc=False, needs_layout_passes=False),
)
def _embed_lookup(table_hbm, idxT_hbm, out_hbm, idx_v, rows0, rows1,
                  tout0, tout1, g0, g1, o0, o1):
    wid = lax.axis_index("s") * NC + lax.axis_index("c")
    a0 = wid * AB
    rows = (rows0, rows1)
    tout = (tout0, tout1)
    gsem = (g0, g1)
    osem = (o0, o1)

    pltpu.sync_copy(idxT_hbm.at[:, pl.ds(a0, AB)], idx_v)

    iota16 = lax.broadcasted_iota(jnp.int32, (L,), 0)

    @plsc.parallel_loop(0, T, unroll=2)
    def _dbl(b):
        for c in range(AB // L):
            sl = pl.ds(c * L, L)
            idx_v[b, sl] = idx_v[b, sl] * 2

    def start_gather(b, p):
        pltpu.async_copy(table_hbm.at[idx_v.at[b]], rows[p], gsem[p])

    def wait_gather(b, p):
        pltpu.make_async_copy(
            table_hbm.at[idx_v.at[b]], rows[p], gsem[p]).wait()

    def start_out(b, p):
        pltpu.async_copy(
            tout[p].at[:, :, pl.ds(0, AB)], out_hbm.at[b, :, wid], osem[p])

    def wait_out(b, p):
        pltpu.make_async_copy(
            tout[p].at[:, :, pl.ds(0, AB)], out_hbm.at[b, :, wid],
            osem[p]).wait()

    dhdl = []
    for j in range(D // L):
        d = j * L + iota16
        dhdl.append((d // 8, d % 8))

    def transpose_scale(p):
        @plsc.parallel_loop(0, AB // 4, unroll=2)
        def _body(i):
            for u in range(4):
                a = i * 4 + u
                cols = jnp.full((L,), a, dtype=jnp.int32)
                for j in range(D // L):
                    vals = rows[p][a, pl.ds(j * L, L)] * SCALE
                    plsc.store_scatter(
                        tout[p], [dhdl[j][0], dhdl[j][1], cols], vals)

    start_gather(0, 0)
    start_gather(1, 1)

    wait_gather(0, 0)
    transpose_scale(0)
    start_out(0, 0)
    wait_gather(1, 1)
    transpose_scale(1)
    start_out(1, 1)

    def superstep(k, carry):
        b0 = 2 * k
        b1 = 2 * k + 1
        wait_out(b0, 0)
        start_gather(b0, 0)
        wait_out(b1, 1)
        start_gather(b1, 1)
        wait_gather(b0, 0)
        transpose_scale(0)
        start_out(b0, 0)
        wait_gather(b1, 1)
        transpose_scale(1)
        start_out(b1, 1)
        return carry

    lax.fori_loop(1, T // 2, superstep, 0)
    wait_out(T - 2, 0)
    wait_out(T - 1, 1)


def kernel(token_ids, embed_table):
    idxT = token_ids.T
    tab2 = jnp.pad(embed_table, ((0, 0), (0, D))).reshape(2 * 1000000, D)
    out5 = _embed_lookup(tab2, idxT)
    out = out5.transpose(2, 4, 0, 1, 3).reshape(S, T, D)
    return out


# speedup vs baseline: 2.5141x; 1.2094x over previous
import functools

import jax
import jax.numpy as jnp
from jax import lax
from jax.experimental import pallas as pl
from jax.experimental.pallas import tpu as pltpu
from jax.experimental.pallas import tpu_sc as plsc

L = 16
NC = 2
NS = 16
NW = NC * NS
S = 4096
T = 200
D = 64
AB = S // NW
TP = AB + 1
SCALE = 8.0

_mesh = plsc.VectorSubcoreMesh(core_axis_name="c", subcore_axis_name="s")


@functools.partial(
    pl.kernel,
    mesh=_mesh,
    out_type=jax.ShapeDtypeStruct((T, D // 8, S // AB, 8, AB), jnp.float32),
    scratch_types=[
        pltpu.VMEM((T, AB), jnp.int32),
        pltpu.VMEM((2 * AB, D), jnp.float32),
        pltpu.VMEM((2 * AB, D), jnp.float32),
        pltpu.VMEM((2, D // 8, 8, TP), jnp.float32),
        pltpu.VMEM((2, D // 8, 8, TP), jnp.float32),
        pltpu.SemaphoreType.DMA,
        pltpu.SemaphoreType.DMA,
        pltpu.SemaphoreType.DMA,
        pltpu.SemaphoreType.DMA,
    ],
    compiler_params=pltpu.CompilerParams(
        use_tc_tiling_on_sc=False, needs_layout_passes=False),
)
def _embed_lookup(table_hbm, idxT_hbm, out_hbm, idx_v, rows0, rows1,
                  tout0, tout1, g0, g1, o0, o1):
    wid = lax.axis_index("s") * NC + lax.axis_index("c")
    a0 = wid * AB
    rows = (rows0, rows1)
    tout = (tout0, tout1)
    gsem = (g0, g1)
    osem = (o0, o1)

    pltpu.sync_copy(idxT_hbm.at[:, pl.ds(a0, AB)], idx_v)

    iota16 = lax.broadcasted_iota(jnp.int32, (L,), 0)

    @plsc.parallel_loop(0, T, unroll=2)
    def _dbl(b):
        for c in range(AB // L):
            sl = pl.ds(c * L, L)
            idx_v[b, sl] = idx_v[b, sl] * 2

    def start_gather(b, p):
        pltpu.async_copy(
            table_hbm.at[idx_v.at[b]], rows[p].at[pl.ds(0, AB)], gsem[p])
        pltpu.async_copy(
            table_hbm.at[idx_v.at[b + 1]], rows[p].at[pl.ds(AB, AB)],
            gsem[p])

    def wait_gather(b, p):
        pltpu.make_async_copy(
            table_hbm.at[idx_v.at[b]], rows[p].at[pl.ds(0, AB)],
            gsem[p]).wait()
        pltpu.make_async_copy(
            table_hbm.at[idx_v.at[b + 1]], rows[p].at[pl.ds(AB, AB)],
            gsem[p]).wait()

    def start_out(b, p):
        pltpu.async_copy(
            tout[p].at[:, :, :, pl.ds(0, AB)],
            out_hbm.at[pl.ds(b, 2), :, wid], osem[p])

    def wait_out(b, p):
        pltpu.make_async_copy(
            tout[p].at[:, :, :, pl.ds(0, AB)],
            out_hbm.at[pl.ds(b, 2), :, wid], osem[p]).wait()

    dhdl = []
    for j in range(D // L):
        d = j * L + iota16
        dhdl.append((d // 8, d % 8))

    def transpose_scale(p):
        @plsc.parallel_loop(0, 2 * AB // 4, unroll=4)
        def _body(i):
            for u in range(4):
                a = i * 4 + u
                half = a // AB
                cols = jnp.full((L,), a % AB, dtype=jnp.int32)
                halves = jnp.full((L,), half, dtype=jnp.int32)
                for j in range(D // L):
                    vals = rows[p][a, pl.ds(j * L, L)] * SCALE
                    plsc.store_scatter(
                        tout[p], [halves, dhdl[j][0], dhdl[j][1], cols],
                        vals)

    start_gather(0, 0)
    start_gather(2, 1)

    wait_gather(0, 0)
    transpose_scale(0)
    start_out(0, 0)
    wait_gather(2, 1)
    transpose_scale(1)
    start_out(2, 1)

    def superstep(k, carry):
        b0 = 4 * k
        b1 = 4 * k + 2
        wait_out(b0, 0)
        start_gather(b0, 0)
        wait_out(b1, 1)
        start_gather(b1, 1)
        wait_gather(b0, 0)
        transpose_scale(0)
        start_out(b0, 0)
        wait_gather(b1, 1)
        transpose_scale(1)
        start_out(b1, 1)
        return carry

    lax.fori_loop(1, T // 4, superstep, 0)
    wait_out(T - 4, 0)
    wait_out(T - 2, 1)


def kernel(token_ids, embed_table):
    idxT = token_ids.T
    tab2 = jnp.pad(embed_table, ((0, 0), (0, D))).reshape(2 * 1000000, D)
    out5 = _embed_lookup(tab2, idxT)
    out = out5.transpose(2, 4, 0, 1, 3).reshape(S, T, D)
    return out
